# Initial kernel scaffold; baseline (speedup 1.0000x reference)
import jax
import jax.numpy as jnp
from jax import lax
from jax.experimental import pallas as pl
from jax.experimental.pallas import tpu as pltpu
from jax.experimental.pallas import tpu_sc as plsc

_NUM_CORES = 2
_NUM_SUBCORES = 16
_NUM_WORKERS = _NUM_CORES * _NUM_SUBCORES

_IDX_BLK = 128
_CHUNK = 1024
_SUB = _CHUNK // _IDX_BLK


def _embed_kernel(idx_hbm, table_hbm, out_hbm, idx_v, rows_v, gsem):
  b_per_w = idx_v.shape[0] * idx_v.shape[1]
  n_chunks = b_per_w // _CHUNK
  wid = lax.axis_index("s") * _NUM_CORES + lax.axis_index("c")
  row_base = wid * (b_per_w // _IDX_BLK)

  pltpu.sync_copy(idx_hbm.at[pl.ds(row_base, b_per_w // _IDX_BLK)], idx_v)

  @pl.loop(0, n_chunks)
  def _chunk(c):
    copies = []
    for j in range(_SUB):
      copies.append(
          pltpu.async_copy(
              table_hbm.at[idx_v.at[c * _SUB + j]],
              rows_v.at[pl.ds(j * _IDX_BLK, _IDX_BLK)],
              gsem,
          )
      )
    for cp in copies:
      cp.wait()
    out_base = wid * b_per_w + c * _CHUNK
    pltpu.sync_copy(rows_v, out_hbm.at[pl.ds(out_base, _CHUNK)])


def kernel(token_ids, weights):
  B0, S = token_ids.shape
  V, D = weights.shape
  B = B0 * S
  assert B % (_NUM_WORKERS * _CHUNK) == 0
  b_per_w = B // _NUM_WORKERS

  idx2d = token_ids.reshape(B // _IDX_BLK, _IDX_BLK).astype(jnp.int32)

  mesh = plsc.VectorSubcoreMesh(core_axis_name="c", subcore_axis_name="s")
  run = pl.kernel(
      _embed_kernel,
      out_type=jax.ShapeDtypeStruct((B, D), jnp.float32),
      mesh=mesh,
      scratch_types=[
          pltpu.VMEM((b_per_w // _IDX_BLK, _IDX_BLK), jnp.int32),
          pltpu.VMEM((_CHUNK, D), jnp.float32),
          pltpu.SemaphoreType.DMA,
      ],
  )
  out = run(idx2d, weights)
  return out.reshape(B0, S, D)


# speedup vs baseline: 1.1027x; 1.1027x over previous
import jax
import jax.numpy as jnp
from jax import lax
from jax.experimental import pallas as pl
from jax.experimental.pallas import tpu as pltpu
from jax.experimental.pallas import tpu_sc as plsc

_NUM_CORES = 2
_NUM_SUBCORES = 16
_NUM_WORKERS = _NUM_CORES * _NUM_SUBCORES

_IDX_BLK = 128
_CHUNK = 1024
_SUB = _CHUNK // _IDX_BLK


def _embed_kernel(idx_hbm, table_hbm, out_hbm, idx_v, rows_v, gsem):
  b_per_w = idx_v.shape[0] * idx_v.shape[1]
  n_chunks = b_per_w // _CHUNK
  wid = lax.axis_index("s") * _NUM_CORES + lax.axis_index("c")
  row_base = wid * (b_per_w // _IDX_BLK)

  pltpu.sync_copy(idx_hbm.at[pl.ds(row_base, b_per_w // _IDX_BLK)], idx_v)

  @pl.loop(0, n_chunks)
  def _chunk(c):
    copies = []
    for j in range(_SUB):
      copies.append(
          pltpu.async_copy(
              table_hbm.at[idx_v.at[c * _SUB + j]],
              rows_v.at[pl.ds(j * _IDX_BLK, _IDX_BLK)],
              gsem,
          )
      )
    for cp in copies:
      cp.wait()
    out_base = wid * b_per_w + c * _CHUNK
    pltpu.sync_copy(rows_v, out_hbm.at[pl.ds(out_base, _CHUNK)])


def kernel(token_ids, weights):
  B0, S = token_ids.shape
  V, D = weights.shape
  B = B0 * S
  assert B % (_NUM_WORKERS * _CHUNK) == 0
  b_per_w = B // _NUM_WORKERS

  idx2d = token_ids.reshape(B // _IDX_BLK, _IDX_BLK).astype(jnp.int32)

  mesh = plsc.VectorSubcoreMesh(core_axis_name="c", subcore_axis_name="s")
  run = pl.kernel(
      _embed_kernel,
      out_type=jax.ShapeDtypeStruct((B, D), jnp.float32),
      mesh=mesh,
      scratch_types=[
          pltpu.VMEM((b_per_w // _IDX_BLK, _IDX_BLK), jnp.int32),
          pltpu.VMEM((_CHUNK, D), jnp.float32),
          pltpu.SemaphoreType.DMA,
      ],
      compiler_params=pltpu.CompilerParams(use_tc_tiling_on_sc=False),
  )
  out = run(idx2d, weights)
  return out.reshape(B0, S, D)


# speedup vs baseline: 1.1128x; 1.0091x over previous
import jax
import jax.numpy as jnp
from jax import lax
from jax.experimental import pallas as pl
from jax.experimental.pallas import tpu as pltpu
from jax.experimental.pallas import tpu_sc as plsc

_NUM_CORES = 2
_NUM_SUBCORES = 16
_NUM_WORKERS = _NUM_CORES * _NUM_SUBCORES

_IDX_BLK = 128
_CHUNK = 1280
_SUB = _CHUNK // _IDX_BLK


def _embed_kernel(idx_hbm, table_hbm, out_hbm, idx_v, rows_v, gsem0, gsem1):
  b_per_w = idx_v.shape[0] * idx_v.shape[1]
  n_chunks = b_per_w // _CHUNK
  wid = lax.axis_index("s") * _NUM_CORES + lax.axis_index("c")
  row_base = wid * (b_per_w // _IDX_BLK)
  gsems = (gsem0, gsem1)

  pltpu.sync_copy(idx_hbm.at[pl.ds(row_base, b_per_w // _IDX_BLK)], idx_v)

  def fire(c, b):
    return [
        pltpu.async_copy(
            table_hbm.at[idx_v.at[c * _SUB + j]],
            rows_v.at[b, pl.ds(j * _IDX_BLK, _IDX_BLK)],
            gsems[b],
        )
        for j in range(_SUB)
    ]

  fire(0, 0)

  @pl.loop(0, n_chunks, step=2)
  def _pair(c):
    for b in range(2):
      cc = c + b
      @pl.when(cc + 1 < n_chunks)
      def _():
        fire(cc + 1, 1 - b)
      for j in range(_SUB):
        pltpu.make_async_copy(
            table_hbm.at[idx_v.at[j]],
            rows_v.at[b, pl.ds(j * _IDX_BLK, _IDX_BLK)],
            gsems[b],
        ).wait()
      out_base = wid * b_per_w + cc * _CHUNK
      pltpu.sync_copy(rows_v.at[b], out_hbm.at[pl.ds(out_base, _CHUNK)])


def kernel(token_ids, weights):
  B0, S = token_ids.shape
  V, D = weights.shape
  B = B0 * S
  assert B % (_NUM_WORKERS * _CHUNK) == 0
  b_per_w = B // _NUM_WORKERS

  idx2d = token_ids.reshape(B // _IDX_BLK, _IDX_BLK).astype(jnp.int32)

  mesh = plsc.VectorSubcoreMesh(core_axis_name="c", subcore_axis_name="s")
  run = pl.kernel(
      _embed_kernel,
      out_type=jax.ShapeDtypeStruct((B, D), jnp.float32),
      mesh=mesh,
      scratch_types=[
          pltpu.VMEM((b_per_w // _IDX_BLK, _IDX_BLK), jnp.int32),
          pltpu.VMEM((2, _CHUNK, D), jnp.float32),
          pltpu.SemaphoreType.DMA,
          pltpu.SemaphoreType.DMA,
      ],
      compiler_params=pltpu.CompilerParams(use_tc_tiling_on_sc=False),
  )
  out = run(idx2d, weights)
  return out.reshape(B0, S, D)


# speedup vs baseline: 1.8062x; 1.6232x over previous
import jax
import jax.numpy as jnp
from jax import lax
from jax.experimental import pallas as pl
from jax.experimental.pallas import tpu as pltpu
from jax.experimental.pallas import tpu_sc as plsc

_NUM_CORES = 2
_NUM_SUBCORES = 16
_NUM_WORKERS = _NUM_CORES * _NUM_SUBCORES

_CB = 16


def _embed_kernel(idx_hbm, table_hbm, out_hbm, idx_v, rows_v, gsem0, gsem1):
  rows_per_w = idx_v.shape[0]
  n_chunks = rows_per_w // _CB
  wid = lax.axis_index("s") * _NUM_CORES + lax.axis_index("c")
  row_base = wid * rows_per_w
  gsems = (gsem0, gsem1)

  pltpu.sync_copy(idx_hbm.at[pl.ds(row_base, rows_per_w)], idx_v)

  def fire(c, b):
    for k in range(_CB):
      pltpu.async_copy(
          table_hbm.at[idx_v.at[c * _CB + k]],
          rows_v.at[b, k],
          gsems[b],
      )

  def drain(b):
    for k in range(_CB):
      pltpu.make_async_copy(
          table_hbm.at[idx_v.at[k]],
          rows_v.at[b, k],
          gsems[b],
      ).wait()

  fire(0, 0)

  @pl.loop(0, n_chunks, step=2)
  def _pair(c):
    for b in range(2):
      cc = c + b
      @pl.when(cc + 1 < n_chunks)
      def _():
        fire(cc + 1, 1 - b)
      drain(b)
      pltpu.sync_copy(rows_v.at[b], out_hbm.at[pl.ds(row_base + cc * _CB, _CB)])


def kernel(token_ids, weights):
  B0, S = token_ids.shape
  V, D = weights.shape
  assert B0 % (_NUM_WORKERS * _CB) == 0
  rows_per_w = B0 // _NUM_WORKERS

  mesh = plsc.VectorSubcoreMesh(core_axis_name="c", subcore_axis_name="s")
  run = pl.kernel(
      _embed_kernel,
      out_type=jax.ShapeDtypeStruct((B0, S, D), jnp.float32),
      mesh=mesh,
      scratch_types=[
          pltpu.VMEM((rows_per_w, S), jnp.int32),
          pltpu.VMEM((2, _CB, S, D), jnp.float32),
          pltpu.SemaphoreType.DMA,
          pltpu.SemaphoreType.DMA,
      ],
      compiler_params=pltpu.CompilerParams(use_tc_tiling_on_sc=False),
  )
  return run(token_ids.astype(jnp.int32), weights)
